# baseline (speedup 1.0000x reference)
import jax
import jax.numpy as jnp
from jax import lax
from jax.experimental import pallas as pl
from jax.experimental.pallas import tpu as pltpu
from jax.experimental.pallas import tpu_sc as plsc

N = 10000
E = 320000
DIN = 128
H1 = 256
H2 = 192
C = 40

NC = 2
NS = 16
CH = 128
CHUNKS = 158
EPT = CHUNKS * CH
EPAD = NS * EPT
NP = 10112
RPT = NP // NS
ZR = 64

DH1 = 72
DH2 = 96
DH3 = 24


def _make_segsum(DH, NBUF):
  HALF = NBUF // 2
  TAIL = CHUNKS % NBUF
  LASTR = N - (NS - 1) * RPT
  mesh = plsc.VectorSubcoreMesh(core_axis_name="c", subcore_axis_name="s")

  def body(u_hbm, packed_hbm, z_hbm, out_hbm, *scratch):
    packed_v = scratch[0]
    src_i = scratch[1:1 + NBUF]
    dst_i = scratch[1 + NBUF:1 + 2 * NBUF]
    rows = scratch[1 + 2 * NBUF:1 + 3 * NBUF]
    agg_sh = scratch[1 + 3 * NBUF]
    sg = scratch[2 + 3 * NBUF:2 + 4 * NBUF]
    ss = scratch[2 + 4 * NBUF:2 + 5 * NBUF]
    c = lax.axis_index("c")
    tid = lax.axis_index("s")
    for i in range(RPT // ZR):
      pltpu.sync_copy(z_hbm, agg_sh.at[pl.ds(tid * RPT + i * ZR, ZR)])
    rem = RPT % ZR
    if rem:
      pltpu.sync_copy(z_hbm.at[pl.ds(0, rem)],
                      agg_sh.at[pl.ds(tid * RPT + (RPT // ZR) * ZR, rem)])
    pltpu.sync_copy(packed_hbm.at[tid], packed_v)
    plsc.subcore_barrier()

    def load_idx(jj, b):
      row = packed_v.at[jj]
      for k in range(CH // 16):
        v = row[pl.ds(k * 16, 16)]
        src_i[b][pl.ds(k * 16, 16)] = v & 0x3FFF
        dst_i[b][pl.ds(k * 16, 16)] = lax.shift_right_logical(v, 14)

    def issue_gather(jj, b):
      load_idx(jj, b)
      pltpu.async_copy(u_hbm.at[c].at[src_i[b]], rows[b], sg[b])

    for b in range(HALF):
      issue_gather(b, b)

    def outer(t, carry):
      for b in range(NBUF):
        jj = t * NBUF + b
        b2 = (b + HALF) % NBUF

        @pl.when(jj >= NBUF - HALF)
        def _(b2=b2):
          pltpu.make_async_copy(
              rows[b2], agg_sh.at[dst_i[b2]], ss[b2]).wait()

        issue_gather(jj + HALF, b2)

        pltpu.make_async_copy(u_hbm.at[c].at[src_i[b]], rows[b], sg[b]).wait()
        pltpu.async_copy(rows[b], agg_sh.at[dst_i[b]], ss[b], add=True)
      return carry

    lax.fori_loop(0, CHUNKS // NBUF, outer, 0)
    for jj in range(CHUNKS - TAIL, CHUNKS):
      b = jj % NBUF
      b2 = (b + HALF) % NBUF
      if jj + HALF < CHUNKS:
        pltpu.make_async_copy(rows[b2], agg_sh.at[dst_i[b2]], ss[b2]).wait()
        issue_gather(jj + HALF, b2)
      pltpu.make_async_copy(u_hbm.at[c].at[src_i[b]], rows[b], sg[b]).wait()
      pltpu.async_copy(rows[b], agg_sh.at[dst_i[b]], ss[b], add=True)
    for b in range(NBUF):
      pltpu.make_async_copy(rows[b], agg_sh.at[dst_i[b]], ss[b]).wait()
    plsc.subcore_barrier()
    @pl.when(tid < NS - 1)
    def _():
      pltpu.sync_copy(agg_sh.at[pl.ds(tid * RPT, RPT)],
                      out_hbm.at[c, pl.ds(tid * RPT, RPT)])

    @pl.when(tid == NS - 1)
    def _():
      pltpu.sync_copy(agg_sh.at[pl.ds((NS - 1) * RPT, LASTR)],
                      out_hbm.at[c, pl.ds((NS - 1) * RPT, LASTR)])

  return pl.kernel(
      body,
      out_type=jax.ShapeDtypeStruct((NC, N, DH), jnp.float32),
      mesh=mesh,
      compiler_params=pltpu.CompilerParams(use_tc_tiling_on_sc=False),
      scratch_types=(
          [pltpu.VMEM((CHUNKS, CH), jnp.int32)]
          + [pltpu.VMEM((CH,), jnp.int32) for _ in range(2 * NBUF)]
          + [pltpu.VMEM((CH, DH), jnp.float32) for _ in range(NBUF)]
          + [pltpu.VMEM_SHARED((NP, DH), jnp.float32)]
          + [pltpu.SemaphoreType.DMA for _ in range(2 * NBUF)]
      ),
  )


_segsum_d1 = _make_segsum(DH1, 4)
_segsum_d2 = _make_segsum(DH2, 3)
_segsum_d3 = _make_segsum(DH3, 8)

BN = 1000


def _self_body(h_ref, w_ref, b_ref, out_ref):
  out_ref[:, :] = (
      jnp.dot(h_ref[:, :], w_ref[:, :], preferred_element_type=jnp.float32)
      + b_ref[:, :])


def _c1_body(p0_ref, p1_ref, s1_ref, wl1_ref, wl2a_ref, wl2b_ref,
             h1_ref, u2_ref, inv_ref):
  s = jnp.concatenate([p0_ref[0], p1_ref[0]], axis=1)
  cnt = s[:, DH1 - 1:DH1]
  inv = 1.0 / jnp.maximum(cnt, 1.0)
  mean = jnp.concatenate(
      [s[:, :DH1 - 1], s[:, DH1:DH1 + DIN - DH1 + 1]], axis=1) * inv
  h1 = jnp.tanh(
      jnp.dot(mean.astype(jnp.bfloat16), wl1_ref[:, :].astype(jnp.bfloat16),
              preferred_element_type=jnp.float32)
      + s1_ref[:, :])
  h1_ref[:, :] = h1
  h1b = h1.astype(jnp.bfloat16)
  u2_ref[0, :, :] = jnp.dot(h1b, wl2a_ref[:, :].astype(jnp.bfloat16),
                            preferred_element_type=jnp.float32)
  u2_ref[1, :, :] = jnp.dot(h1b, wl2b_ref[:, :].astype(jnp.bfloat16),
                            preferred_element_type=jnp.float32)
  inv_ref[:, :] = jnp.broadcast_to(inv, (BN, 8))


def _c2_body(p0_ref, p1_ref, s2_ref, inv_ref, wl3a_ref, wl3b_ref,
             h2_ref, u3_ref):
  agg = jnp.concatenate([p0_ref[0], p1_ref[0]], axis=1) * inv_ref[:, 0:1]
  h2 = jax.nn.relu(agg + s2_ref[:, :])
  h2_ref[:, :] = h2
  h2b = h2.astype(jnp.bfloat16)
  u3_ref[0, :, :] = jnp.dot(h2b, wl3a_ref[:, :].astype(jnp.bfloat16),
                            preferred_element_type=jnp.float32)
  u3_ref[1, :, :] = jnp.dot(h2b, wl3b_ref[:, :].astype(jnp.bfloat16),
                            preferred_element_type=jnp.float32)


def _c3_body(p0_ref, p1_ref, s3_ref, inv_ref, out_ref):
  agg = jnp.concatenate([p0_ref[0], p1_ref[0][:, :C - DH3]], axis=1)
  out_ref[:, :] = jax.nn.sigmoid(agg * inv_ref[:, 0:1] + s3_ref[:, :])


def _row_spec(d):
  return pl.BlockSpec((BN, d), lambda i: (i, 0))


def _core_spec(d, k):
  return pl.BlockSpec((1, BN, d), lambda i, k=k: (k, i, 0))


def _stack_spec(d):
  return pl.BlockSpec((NC, BN, d), lambda i: (0, i, 0))


def _full_spec(shape):
  return pl.BlockSpec(shape, lambda i: (0,) * len(shape))


def _make_self(din, dout):
  return pl.pallas_call(
      _self_body,
      grid=(N // BN,),
      in_specs=[_row_spec(din), _full_spec((din, dout)),
                _full_spec((1, dout))],
      out_specs=_row_spec(dout),
      out_shape=jax.ShapeDtypeStruct((N, dout), jnp.float32),
  )


_self1 = _make_self(DIN, H1)
_self2 = _make_self(H1, H2)
_self3 = _make_self(H2, C)

_combine1 = pl.pallas_call(
    _c1_body,
    grid=(N // BN,),
    in_specs=[
        _core_spec(DH1, 0), _core_spec(DH1, 1), _row_spec(H1),
        _full_spec((DIN, H1)),
        _full_spec((H1, DH2)), _full_spec((H1, DH2)),
    ],
    out_specs=[_row_spec(H1), _stack_spec(DH2), _row_spec(8)],
    out_shape=[
        jax.ShapeDtypeStruct((N, H1), jnp.float32),
        jax.ShapeDtypeStruct((NC, N, DH2), jnp.float32),
        jax.ShapeDtypeStruct((N, 8), jnp.float32),
    ],
)

_combine2 = pl.pallas_call(
    _c2_body,
    grid=(N // BN,),
    in_specs=[
        _core_spec(DH2, 0), _core_spec(DH2, 1), _row_spec(H2), _row_spec(8),
        _full_spec((H2, DH3)), _full_spec((H2, DH3)),
    ],
    out_specs=[_row_spec(H2), _stack_spec(DH3)],
    out_shape=[
        jax.ShapeDtypeStruct((N, H2), jnp.float32),
        jax.ShapeDtypeStruct((NC, N, DH3), jnp.float32),
    ],
)

_combine3 = pl.pallas_call(
    _c3_body,
    grid=(N // BN,),
    in_specs=[
        _core_spec(DH3, 0), _core_spec(DH3, 1), _row_spec(C), _row_spec(8),
    ],
    out_specs=_row_spec(C),
    out_shape=jax.ShapeDtypeStruct((N, C), jnp.float32),
)


def kernel(x, edge_index, batch, Wl1, b1, Wr1, Wl2, b2, Wr2, Wl3, b3, Wr3):
  f32 = jnp.float32
  src = edge_index[0]
  dst = edge_index[1]
  pad = EPAD - E
  packed = jnp.left_shift(dst, 14) | src
  packedp = jnp.concatenate(
      [packed, jnp.full((pad,), N << 14, jnp.int32)]).reshape(NS, CHUNKS, CH)

  xa = jnp.stack([
      jnp.concatenate([x[:, :DH1 - 1], jnp.ones((N, 1), f32)], axis=1),
      jnp.concatenate([x[:, DH1 - 1:DIN],
                       jnp.zeros((N, 2 * DH1 - DIN - 1), f32)], axis=1),
  ])
  p1 = _segsum_d1(xa, packedp, jnp.zeros((ZR, DH1), f32))
  s1 = _self1(x, Wr1, b1.reshape(1, H1))
  h1, u2, inv = _combine1(p1, p1, s1, Wl1, Wl2[:, :DH2], Wl2[:, DH2:])

  p2 = _segsum_d2(u2, packedp, jnp.zeros((ZR, DH2), f32))
  s2 = _self2(h1, Wr2, b2.reshape(1, H2))
  wl3p = jnp.pad(Wl3, ((0, 0), (0, 2 * DH3 - C)))
  h2, u3 = _combine2(p2, p2, s2, inv, wl3p[:, :DH3], wl3p[:, DH3:])

  p3 = _segsum_d3(u3, packedp, jnp.zeros((ZR, DH3), f32))
  s3 = _self3(h2, Wr3, b3.reshape(1, C))
  return _combine3(p3, p3, s3, inv)
